# Initial kernel scaffold; baseline (speedup 1.0000x reference)
import jax
import jax.numpy as jnp
from jax.experimental import pallas as pl


def kernel(x, edge_indices, edge_attrs, s_local, Theta0, Theta1, bias):
    raise NotImplementedError("write your pallas kernel here")


# speedup vs baseline: 2.9002x; 2.9002x over previous
import functools

import jax
import jax.numpy as jnp
from jax import lax
from jax.experimental import pallas as pl
from jax.experimental.pallas import tpu as pltpu
from jax.experimental.pallas import tpu_sc as plsc

N = 10000
E = 320000
F_IN = 128
F_OUT = 32
FC = 2 * F_OUT

NC, NS = 2, 16
NW = NC * NS
NPAD = 10240
EPAD = 327680
EPT = EPAD // NW
BLK = 128
NBLK = EPT // BLK
ROWS_PT = NPAD // NS

BR = 1024


def _sc_round_body(h_hbm, src_hbm, dst_hbm, w_hbm, out_hbm,
                   src_v, dst_v, w_v, rows_v, acc_sh, sem):
    c = lax.axis_index("c")
    s = lax.axis_index("s")
    wid = c * NS + s

    def _zrow(i, _):
        for f in range(FC // 16):
            rows_v[i, pl.ds(f * 16, 16)] = jnp.zeros((16,), jnp.float32)
        return _
    lax.fori_loop(0, BLK, _zrow, 0, unroll=8)
    row0 = s * ROWS_PT
    for zb in range(ROWS_PT // BLK):
        pltpu.sync_copy(rows_v, acc_sh.at[pl.ds(row0 + zb * BLK, BLK)])
    plsc.subcore_barrier()

    ebase = wid * EPT

    def _block(blk, _):
        off = ebase + blk * BLK
        pltpu.sync_copy(src_hbm.at[pl.ds(off, BLK)], src_v)
        pltpu.sync_copy(dst_hbm.at[pl.ds(off, BLK)], dst_v)
        pltpu.sync_copy(w_hbm.at[pl.ds(off, BLK)], w_v)
        pltpu.async_copy(h_hbm.at[src_v], rows_v, sem).wait()

        def _scale(g, _):
            wv = w_v[pl.ds(g * 16, 16)]
            for j in range(16):
                e = g * 16 + j
                wspl = jnp.full((16,), wv[j])
                for f in range(FC // 16):
                    sl = pl.ds(f * 16, 16)
                    rows_v[e, sl] = rows_v[e, sl] * wspl
            return _
        lax.fori_loop(0, BLK // 16, _scale, 0)

        pltpu.sync_copy(rows_v, acc_sh.at[dst_v], add=True)
        return _
    lax.fori_loop(0, NBLK, _block, 0)

    plsc.subcore_barrier()
    pltpu.sync_copy(acc_sh.at[pl.ds(row0, ROWS_PT)],
                    out_hbm.at[c, pl.ds(row0, ROWS_PT)])


_sc_round = functools.partial(
    pl.kernel,
    out_type=jax.ShapeDtypeStruct((NC, NPAD, FC), jnp.float32),
    mesh=plsc.VectorSubcoreMesh(core_axis_name="c", subcore_axis_name="s",
                                num_cores=NC, num_subcores=NS),
    compiler_params=pltpu.CompilerParams(use_tc_tiling_on_sc=False),
    scratch_types=[
        pltpu.VMEM((BLK,), jnp.int32),
        pltpu.VMEM((BLK,), jnp.int32),
        pltpu.VMEM((BLK,), jnp.float32),
        pltpu.VMEM((BLK, FC), jnp.float32),
        pltpu.VMEM_SHARED((NPAD, FC), jnp.float32),
        pltpu.SemaphoreType.DMA,
    ],
)(_sc_round_body)


def _project_body(x_ref, th_ref, b_ref, a1_ref, a2_ref, a3_ref, bc_ref):
    z = jnp.dot(x_ref[...], th_ref[...], preferred_element_type=jnp.float32)
    a1_ref[...] = z[:, FC:2 * FC]
    a2_ref[...] = z[:, 2 * FC:3 * FC]
    a3_ref[...] = z[:, 3 * FC:4 * FC]
    bc_ref[...] = z[:, 0:FC] - z[:, 2 * FC:3 * FC] + b_ref[...]


def _project(x_pad, theta_cat, bias_cat):
    grid = (NPAD // BR,)
    return pl.pallas_call(
        _project_body,
        grid=grid,
        in_specs=[
            pl.BlockSpec((BR, F_IN), lambda i: (i, 0)),
            pl.BlockSpec((F_IN, 4 * FC), lambda i: (0, 0)),
            pl.BlockSpec((1, FC), lambda i: (0, 0)),
        ],
        out_specs=[pl.BlockSpec((BR, FC), lambda i: (i, 0))] * 4,
        out_shape=[jax.ShapeDtypeStruct((NPAD, FC), jnp.float32)] * 4,
    )(x_pad, theta_cat, bias_cat)


def _combine1_body(u_ref, a2_ref, o_ref):
    o_ref[...] = a2_ref[...] + 2.0 * (u_ref[0] + u_ref[1])


def _combine1(u, a2):
    return pl.pallas_call(
        _combine1_body,
        grid=(NPAD // BR,),
        in_specs=[
            pl.BlockSpec((NC, BR, FC), lambda i: (0, i, 0)),
            pl.BlockSpec((BR, FC), lambda i: (i, 0)),
        ],
        out_specs=pl.BlockSpec((BR, FC), lambda i: (i, 0)),
        out_shape=jax.ShapeDtypeStruct((NPAD, FC), jnp.float32),
    )(u, a2)


def _combine2_body(v_ref, a1_ref, a3_ref, o_ref):
    o_ref[...] = a1_ref[...] - 3.0 * a3_ref[...] + 2.0 * (v_ref[0] + v_ref[1])


def _combine2(v, a1, a3):
    return pl.pallas_call(
        _combine2_body,
        grid=(NPAD // BR,),
        in_specs=[
            pl.BlockSpec((NC, BR, FC), lambda i: (0, i, 0)),
            pl.BlockSpec((BR, FC), lambda i: (i, 0)),
            pl.BlockSpec((BR, FC), lambda i: (i, 0)),
        ],
        out_specs=pl.BlockSpec((BR, FC), lambda i: (i, 0)),
        out_shape=jax.ShapeDtypeStruct((NPAD, FC), jnp.float32),
    )(v, a1, a3)


def _epilogue_body(r_ref, bc_ref, sb_ref, o_ref):
    p = bc_ref[...] + r_ref[0] + r_ref[1]
    o_ref[...] = p[:, :F_OUT] + sb_ref[...] * p[:, F_OUT:]


def _epilogue(r, bc, sb):
    return pl.pallas_call(
        _epilogue_body,
        grid=(NPAD // BR,),
        in_specs=[
            pl.BlockSpec((NC, BR, FC), lambda i: (0, i, 0)),
            pl.BlockSpec((BR, FC), lambda i: (i, 0)),
            pl.BlockSpec((BR, F_OUT), lambda i: (i, 0)),
        ],
        out_specs=pl.BlockSpec((BR, F_OUT), lambda i: (i, 0)),
        out_shape=jax.ShapeDtypeStruct((NPAD, F_OUT), jnp.float32),
    )(r, bc, sb)


@jax.jit
def kernel(x, edge_indices, edge_attrs, s_local, Theta0, Theta1, bias):
    x_pad = jnp.pad(x[0], ((0, NPAD - N), (0, 0)))
    src = jnp.pad(edge_indices[0], (0, EPAD - E))
    dst = jnp.pad(edge_indices[1], (0, EPAD - E))
    w = jnp.pad(edge_attrs, (0, EPAD - E))
    sb = jnp.broadcast_to(jnp.pad(s_local[0], (0, NPAD - N))[:, None],
                          (NPAD, F_OUT))

    theta_cat = jnp.concatenate(
        [jnp.concatenate([Theta0[k], Theta1[k]], axis=1) for k in range(4)],
        axis=1)
    bias_cat = jnp.concatenate([bias, jnp.zeros_like(bias)])[None, :]

    a1, a2, a3, bc = _project(x_pad, theta_cat, bias_cat)

    u = _sc_round(a3, src, dst, w)
    h2 = _combine1(u, a2)
    v = _sc_round(h2, src, dst, w)
    h3 = _combine2(v, a1, a3)
    r = _sc_round(h3, src, dst, w)

    out = _epilogue(r, bc, sb)
    return out[:N][None, :, :]
